# Initial kernel scaffold; baseline (speedup 1.0000x reference)
import jax
import jax.numpy as jnp
from jax.experimental import pallas as pl


def kernel(x, edge_index, edge_weight, W1, b1, W2, b2):
    raise NotImplementedError("write your pallas kernel here")


# speedup vs baseline: 3.6496x; 3.6496x over previous
import functools

import jax
import jax.numpy as jnp
from jax import lax
from jax.experimental import pallas as pl
from jax.experimental.pallas import tpu as pltpu
from jax.experimental.pallas import tpu_sc as plsc

NC = 2
NS = 16
LANES = 16
CHUNK = 128


def _sc_spmm(xmat, src, dst, w, n_chunks):
    n, d = xmat.shape
    n_per_tile = n_chunks * CHUNK
    nvreg = d // LANES
    rows_per_tile = (n + NS - 1) // NS
    mesh = plsc.VectorSubcoreMesh(
        core_axis_name="c", subcore_axis_name="s",
        num_cores=NC, num_subcores=NS)

    @functools.partial(
        pl.kernel, mesh=mesh,
        out_type=jax.ShapeDtypeStruct((NC, n, d), jnp.float32),
        scratch_types=[
            pltpu.VMEM((CHUNK,), jnp.int32),
            pltpu.VMEM((CHUNK,), jnp.int32),
            pltpu.VMEM((CHUNK,), jnp.float32),
            pltpu.VMEM((CHUNK, d), jnp.float32),
            pltpu.VMEM_SHARED((n, d), jnp.float32),
            pltpu.SemaphoreType.DMA,
        ],
    )
    def spmm(x_hbm, src_hbm, dst_hbm, w_hbm, out_hbm, sidx, didx, wv, rows,
             acc, sem):
        c = lax.axis_index("c")
        s = lax.axis_index("s")

        zval = jnp.zeros((LANES,), jnp.float32)

        def zbody(i, _):
            for f in range(nvreg):
                rows[i, pl.ds(f * LANES, LANES)] = zval
            return 0

        lax.fori_loop(0, CHUNK, zbody, 0)

        nrk = -(-n // CHUNK)
        for k in range(nrk):
            size = min(CHUNK, n - k * CHUNK)

            def _zero(off=k * CHUNK, sz=size):
                pltpu.sync_copy(rows.at[pl.ds(0, sz)],
                                acc.at[pl.ds(off, sz)])

            pl.when(jnp.equal(k % NS, s))(_zero)
        plsc.subcore_barrier()

        ebase = (c * NS + s) * n_per_tile

        def chunk_body(g, _):
            base = ebase + g * CHUNK
            pltpu.sync_copy(src_hbm.at[pl.ds(base, CHUNK)], sidx)
            pltpu.sync_copy(dst_hbm.at[pl.ds(base, CHUNK)], didx)
            pltpu.sync_copy(w_hbm.at[pl.ds(base, CHUNK)], wv)
            pltpu.async_copy(x_hbm.at[sidx], rows, sem).wait()

            def scale_body(g, _):
                w16 = wv[pl.ds(g * LANES, LANES)]
                for e in range(LANES):
                    we = w16[e]
                    row = g * LANES + e
                    for f in range(nvreg):
                        sl = pl.ds(f * LANES, LANES)
                        rows[row, sl] = rows[row, sl] * we
                return 0

            lax.fori_loop(0, CHUNK // LANES, scale_body, 0)
            pltpu.sync_copy(rows, acc.at[didx], add=True)
            return 0

        lax.fori_loop(0, n_chunks, chunk_body, 0)
        plsc.subcore_barrier()

        for k in range(nrk):
            size = min(CHUNK, n - k * CHUNK)

            def _flush(off=k * CHUNK, sz=size):
                pltpu.sync_copy(acc.at[pl.ds(off, sz)],
                                out_hbm.at[c, pl.ds(off, sz)])

            pl.when(jnp.equal(k % NS, s))(_flush)

    return spmm(xmat, src, dst, w)


def _tc_layer(z, yp, Wa, Wb, b, relu, block_n):
    n, d = z.shape
    dout = Wa.shape[1]
    grid = n // block_n

    def body(z_ref, yp_ref, wa_ref, wb_ref, b_ref, o_ref):
        ysum = yp_ref[0] + yp_ref[1]
        acc = jnp.dot(z_ref[...], wa_ref[...],
                      preferred_element_type=jnp.float32)
        acc += jnp.dot(ysum, wb_ref[...], preferred_element_type=jnp.float32)
        acc += b_ref[...][None, :]
        if relu:
            acc = jnp.maximum(acc, 0.0)
        o_ref[...] = acc

    return pl.pallas_call(
        body,
        grid=(grid,),
        in_specs=[
            pl.BlockSpec((block_n, d), lambda i: (i, 0)),
            pl.BlockSpec((2, block_n, d), lambda i: (0, i, 0)),
            pl.BlockSpec((d, dout), lambda i: (0, 0)),
            pl.BlockSpec((d, dout), lambda i: (0, 0)),
            pl.BlockSpec((dout,), lambda i: (0,)),
        ],
        out_specs=pl.BlockSpec((block_n, dout), lambda i: (i, 0)),
        out_shape=jax.ShapeDtypeStruct((n, dout), jnp.float32),
    )(z, yp, Wa, Wb, b)


def kernel(x, edge_index, edge_weight, W1, b1, W2, b2):
    n, d = x.shape
    e = edge_weight.shape[0]

    W1a, W1b = W1[0::2], W1[1::2]
    W2a, W2b = W2[0::2], W2[1::2]

    per_tile = -(-e // (NC * NS * CHUNK)) * CHUNK
    n_chunks = per_tile // CHUNK
    e_pad = per_tile * NC * NS
    pad = e_pad - e
    src = jnp.pad(edge_index[1], (0, pad))
    dst = jnp.pad(edge_index[0], (0, pad))
    w = jnp.pad(edge_weight, (0, pad))

    y1 = _sc_spmm(x, src, dst, w, n_chunks)
    h = _tc_layer(x, y1, W1a, W1b, b1, relu=True, block_n=1000)
    y2 = _sc_spmm(h, src, dst, w, n_chunks)
    out = _tc_layer(h, y2, W2a, W2b, b2, relu=False, block_n=1000)
    return out
